# baseline (speedup 1.0000x reference)
import functools

import jax
import jax.numpy as jnp
from jax import lax
from jax.experimental import pallas as pl
from jax.experimental.pallas import tpu as pltpu
from jax.experimental.pallas import tpu_sc as plsc

H = 32
D = 128
HID = 4096
SEQ_PREV = 8191
PAGE = 16
BUDGET = 2048
ROPE_THETA = 10000.0
P = (SEQ_PREV + 1) // PAGE
NSEL = BUDGET // PAGE
SROWS = 256
NSLAB = (SEQ_PREV + 1) // SROWS
NEG = -1e30

INTERPRET = False


def _proj_kernel_rope(x_ref, w_ref, cos_ref, sin_ref, o_ref):
    t = lax.dot_general(x_ref[...], w_ref[...], (((1,), (1,)), ((), ())),
                        preferred_element_type=jnp.float32)
    c = cos_ref[...]
    s = sin_ref[...]
    x1 = t[:, : D // 2]
    x2 = t[:, D // 2:]
    o_ref[0] = jnp.concatenate([x1 * c - x2 * s, x2 * c + x1 * s], axis=1)


def _proj_kernel_plain(x_ref, w_ref, o_ref):
    o_ref[0] = lax.dot_general(x_ref[...], w_ref[...], (((1,), (1,)), ((), ())),
                               preferred_element_type=jnp.float32)


def _proj(x, w, cos=None, sin=None):
    rope = cos is not None
    in_specs = [
        pl.BlockSpec((1, HID), lambda i: (0, 0)),
        pl.BlockSpec((D, HID), lambda i: (i, 0)),
    ]
    args = [x, w]
    if rope:
        in_specs += [pl.BlockSpec((1, D // 2), lambda i: (0, 0)),
                     pl.BlockSpec((1, D // 2), lambda i: (0, 0))]
        args += [cos, sin]
    return pl.pallas_call(
        _proj_kernel_rope if rope else _proj_kernel_plain,
        grid=(HID // D,),
        in_specs=in_specs,
        out_specs=pl.BlockSpec((1, 1, D), lambda i: (i, 0, 0)),
        out_shape=jax.ShapeDtypeStruct((HID // D, 1, D), jnp.float32),
        interpret=INTERPRET,
    )(*args)


def _est_kernel(k_ref, knew_ref, q_ref, est_ref):
    sb = pl.program_id(0)
    blk = k_ref[...]
    row = sb * SROWS + lax.broadcasted_iota(jnp.int32, (SROWS, 1, 1), 0)
    blk = jnp.where(row == SEQ_PREV, knew_ref[...][None], blk)
    pages = blk.reshape(SROWS // PAGE, PAGE, H, D)
    kmin = pages.min(axis=1)
    kmax = pages.max(axis=1)
    q = q_ref[...]
    m = jnp.maximum(q * kmin, q * kmax)
    est_ref[...] = jnp.sum(m, axis=-1, keepdims=True)


def _estimate(k_cache, knew, q):
    return pl.pallas_call(
        _est_kernel,
        grid=(NSLAB,),
        in_specs=[
            pl.BlockSpec((SROWS, H, D), lambda i: (i, 0, 0)),
            pl.BlockSpec((H, D), lambda i: (0, 0)),
            pl.BlockSpec((H, D), lambda i: (0, 0)),
        ],
        out_specs=pl.BlockSpec((SROWS // PAGE, H, 1), lambda i: (i, 0, 0)),
        out_shape=jax.ShapeDtypeStruct((P, H, 1), jnp.float32),
        compiler_params=pltpu.CompilerParams(
            dimension_semantics=("arbitrary",)),
        interpret=INTERPRET,
    )(k_cache, knew, q)


def _cut_kernel(est_ref, bias_ref):
    e = est_ref[...]
    i = lax.bitcast_convert_type(e, jnp.int32)
    enc = jnp.where(i >= 0, i, i ^ jnp.int32(0x7FFFFFFF))
    t = jnp.full((1, H), jnp.int32(-2147483648))
    kk = jnp.int32(NSEL)
    cnt0 = jnp.sum((enc >= 0).astype(jnp.int32), axis=0, keepdims=True)
    t = jnp.where(cnt0 >= kk, jnp.zeros_like(t), t)
    for b in range(30, -1, -1):
        cand = t + jnp.int32(1 << b)
        cnt = jnp.sum((enc >= cand).astype(jnp.int32), axis=0, keepdims=True)
        t = jnp.where(cnt >= kk, cand, t)
    gt = enc > t
    cnt_gt = jnp.sum(gt.astype(jnp.int32), axis=0, keepdims=True)
    need = (kk - cnt_gt).astype(jnp.float32)
    eq = enc == t
    eqf = eq.astype(jnp.float32)
    r = lax.broadcasted_iota(jnp.int32, (P, P), 0)
    c = lax.broadcasted_iota(jnp.int32, (P, P), 1)
    lt = (c <= r).astype(jnp.float32)
    prefix = lax.dot_general(lt, eqf, (((1,), (0,)), ((), ())),
                             preferred_element_type=jnp.float32)
    take = jnp.logical_or(gt, jnp.logical_and(eq, prefix <= need))
    bias_ref[...] = jnp.where(take, 0.0, NEG)


def _page_cut(est):
    return pl.pallas_call(
        _cut_kernel,
        in_specs=[pl.BlockSpec((P, H), lambda: (0, 0))],
        out_specs=pl.BlockSpec((P, H), lambda: (0, 0)),
        out_shape=jax.ShapeDtypeStruct((P, H), jnp.float32),
        interpret=INTERPRET,
    )(est)


_SC_L = 16


def _sc_cut_body(bits_hbm, bias_hbm, bits_v, enc_v, bias_v):
    wid = lax.axis_index("s") * 2 + lax.axis_index("c")
    pltpu.sync_copy(bits_hbm.at[wid], bits_v)
    nchunk = P // _SC_L
    one = jnp.ones((_SC_L,), jnp.int32)
    zer = jnp.zeros((_SC_L,), jnp.int32)

    def enc_chunk(j, _):
        i = bits_v[pl.ds(j * _SC_L, _SC_L)]
        enc_v[pl.ds(j * _SC_L, _SC_L)] = jnp.where(
            i >= 0, i, i ^ jnp.int32(0x7FFFFFFF))
        return 0

    lax.fori_loop(0, nchunk, enc_chunk, 0)

    def count_ge(cand_s):
        cand = jnp.full((_SC_L,), cand_s, jnp.int32)

        def body(j, acc):
            e = enc_v[pl.ds(j * _SC_L, _SC_L)]
            return acc + jnp.where(e >= cand, one, zer)

        acc = lax.fori_loop(0, nchunk, body, jnp.zeros((_SC_L,), jnp.int32))
        tot = jnp.int32(0)
        for l in range(_SC_L):
            tot = tot + acc[l]
        return tot

    kk = jnp.int32(NSEL)
    t = jnp.where(count_ge(jnp.int32(0)) >= kk,
                  jnp.int32(0), jnp.int32(-2147483648))
    for b in range(30, -1, -1):
        cand = t + jnp.int32(1 << b)
        t = jnp.where(count_ge(cand) >= kk, cand, t)
    need = kk - count_ge(t + jnp.int32(1))

    tvec = jnp.full((_SC_L,), t, jnp.int32)
    nvec = jnp.full((_SC_L,), need, jnp.int32)
    ramp = lax.iota(jnp.int32, _SC_L)
    ties = jnp.int32(0)
    for j in range(nchunk):
        e = enc_v[pl.ds(j * _SC_L, _SC_L)]
        eq = e == tvec
        eqi = jnp.where(eq, one, zer)
        pre = jnp.full((_SC_L,), ties, jnp.int32)
        for l in range(_SC_L):
            pre = pre + jnp.where(ramp >= l, eqi[l], jnp.int32(0))
        ties = pre[_SC_L - 1]
        take = jnp.logical_or(e > tvec, jnp.logical_and(eq, pre <= nvec))
        bias_v[pl.ds(j * _SC_L, _SC_L)] = jnp.where(take, 0.0, NEG)
    pltpu.sync_copy(bias_v, bias_hbm.at[wid])


def _page_cut_sc(est_hp):
    bits = lax.bitcast_convert_type(est_hp, jnp.int32)
    fn = functools.partial(
        pl.kernel,
        mesh=plsc.VectorSubcoreMesh(core_axis_name="c", subcore_axis_name="s"),
        out_type=jax.ShapeDtypeStruct((H, P), jnp.float32),
        scratch_types=[
            pltpu.VMEM((P,), jnp.int32),
            pltpu.VMEM((P,), jnp.int32),
            pltpu.VMEM((P,), jnp.float32),
        ],
    )(_sc_cut_body)
    return fn(bits)


def _attn_kernel(k_ref, v_ref, q_ref, knew_ref, vnew_ref, bias_ref, o_ref,
                 acc_ref, m_ref, l_ref):
    i = pl.program_id(0)

    @pl.when(i == 0)
    def _init():
        m_ref[...] = jnp.full((H, 1), NEG)
        l_ref[...] = jnp.zeros((H, 1), jnp.float32)
        acc_ref[...] = jnp.zeros((H, D), jnp.float32)

    kblk = k_ref[...]
    vblk = v_ref[...]
    row = i * SROWS + lax.broadcasted_iota(jnp.int32, (SROWS, 1, 1), 0)
    isnew = row == SEQ_PREV
    kblk = jnp.where(isnew, knew_ref[...][None], kblk)
    vblk = jnp.where(isnew, vnew_ref[...][None], vblk)

    q = q_ref[...]
    s = lax.dot_general(q, kblk, (((1,), (2,)), ((0,), (1,))),
                        preferred_element_type=jnp.float32)
    s = s * (1.0 / (D ** 0.5)) + bias_ref[...]
    m_prev = m_ref[...]
    m_new = jnp.maximum(m_prev, jnp.max(s, axis=1, keepdims=True))
    corr = jnp.exp(m_prev - m_new)
    p = jnp.exp(s - m_new)
    l_ref[...] = l_ref[...] * corr + jnp.sum(p, axis=1, keepdims=True)
    pv = lax.dot_general(p, vblk, (((1,), (0,)), ((0,), (1,))),
                         preferred_element_type=jnp.float32)
    acc_ref[...] = acc_ref[...] * corr + pv
    m_ref[...] = m_new

    @pl.when(i == NSLAB - 1)
    def _fin():
        o_ref[...] = acc_ref[...] / l_ref[...]


def _masked_attn(k_cache, v_cache, q, knew, vnew, bias_rows):
    return pl.pallas_call(
        _attn_kernel,
        grid=(NSLAB,),
        in_specs=[
            pl.BlockSpec((SROWS, H, D), lambda i: (i, 0, 0)),
            pl.BlockSpec((SROWS, H, D), lambda i: (i, 0, 0)),
            pl.BlockSpec((H, D), lambda i: (0, 0)),
            pl.BlockSpec((H, D), lambda i: (0, 0)),
            pl.BlockSpec((H, D), lambda i: (0, 0)),
            pl.BlockSpec((H, SROWS), lambda i: (0, i)),
        ],
        out_specs=pl.BlockSpec((H, D), lambda i: (0, 0)),
        out_shape=jax.ShapeDtypeStruct((H, D), jnp.float32),
        scratch_shapes=[
            pltpu.VMEM((H, D), jnp.float32),
            pltpu.VMEM((H, 1), jnp.float32),
            pltpu.VMEM((H, 1), jnp.float32),
        ],
        compiler_params=pltpu.CompilerParams(
            dimension_semantics=("arbitrary",)),
        interpret=INTERPRET,
    )(k_cache, v_cache, q, knew, vnew, bias_rows)


def kernel(hidden_states, k_cache, v_cache, Wq, Wk, Wv, Wo):
    x = hidden_states.reshape(1, HID)
    d2 = D // 2
    inv_freq = 1.0 / (ROPE_THETA ** (jnp.arange(0, d2, dtype=jnp.float32) / d2))
    ang = jnp.float32(SEQ_PREV) * inv_freq
    cos = jnp.cos(ang).reshape(1, d2)
    sin = jnp.sin(ang).reshape(1, d2)

    q = _proj(x, Wq, cos, sin).reshape(H, D)
    knew = _proj(x, Wk, cos, sin).reshape(H, D)
    vnew = _proj(x, Wv).reshape(H, D)

    est = _estimate(k_cache, knew, q)
    bias = _page_cut_sc(est.reshape(P, H).T)
    bias_rows = jnp.repeat(bias, PAGE, axis=1)

    att = _masked_attn(k_cache, v_cache, q, knew, vnew, bias_rows)

    out = _proj(att.reshape(1, HID), Wo)
    return out.reshape(1, 1, HID)


# speedup vs baseline: 1.6118x; 1.6118x over previous
import functools

import jax
import jax.numpy as jnp
from jax import lax
from jax.experimental import pallas as pl
from jax.experimental.pallas import tpu as pltpu
from jax.experimental.pallas import tpu_sc as plsc

H = 32
D = 128
HID = 4096
SEQ_PREV = 8191
PAGE = 16
BUDGET = 2048
ROPE_THETA = 10000.0
P = (SEQ_PREV + 1) // PAGE
NSEL = BUDGET // PAGE
SROWS = 256
NSLAB = (SEQ_PREV + 1) // SROWS
NEG = -1e30

INTERPRET = False


def _proj_kernel_rope(x_ref, w_ref, cos_ref, sin_ref, o_ref):
    t = lax.dot_general(x_ref[...], w_ref[...], (((1,), (1,)), ((), ())),
                        preferred_element_type=jnp.float32)
    c = cos_ref[...]
    s = sin_ref[...]
    x1 = t[:, : D // 2]
    x2 = t[:, D // 2:]
    o_ref[0] = jnp.concatenate([x1 * c - x2 * s, x2 * c + x1 * s], axis=1)


def _proj_kernel_plain(x_ref, w_ref, o_ref):
    o_ref[0] = lax.dot_general(x_ref[...], w_ref[...], (((1,), (1,)), ((), ())),
                               preferred_element_type=jnp.float32)


def _proj(x, w, cos=None, sin=None):
    rope = cos is not None
    in_specs = [
        pl.BlockSpec((1, HID), lambda i: (0, 0)),
        pl.BlockSpec((D, HID), lambda i: (i, 0)),
    ]
    args = [x, w]
    if rope:
        in_specs += [pl.BlockSpec((1, D // 2), lambda i: (0, 0)),
                     pl.BlockSpec((1, D // 2), lambda i: (0, 0))]
        args += [cos, sin]
    return pl.pallas_call(
        _proj_kernel_rope if rope else _proj_kernel_plain,
        grid=(HID // D,),
        in_specs=in_specs,
        out_specs=pl.BlockSpec((1, 1, D), lambda i: (i, 0, 0)),
        out_shape=jax.ShapeDtypeStruct((HID // D, 1, D), jnp.float32),
        interpret=INTERPRET,
    )(*args)


def _est_kernel(k_ref, knew_ref, q_ref, est_ref):
    sb = pl.program_id(0)
    blk = k_ref[...]
    row = sb * SROWS + lax.broadcasted_iota(jnp.int32, (SROWS, 1, 1), 0)
    blk = jnp.where(row == SEQ_PREV, knew_ref[...][None], blk)
    pages = blk.reshape(SROWS // PAGE, PAGE, H, D)
    kmin = pages.min(axis=1)
    kmax = pages.max(axis=1)
    q = q_ref[...]
    m = jnp.maximum(q * kmin, q * kmax)
    est_ref[...] = jnp.sum(m, axis=-1, keepdims=True)


def _estimate(k_cache, knew, q):
    return pl.pallas_call(
        _est_kernel,
        grid=(NSLAB,),
        in_specs=[
            pl.BlockSpec((SROWS, H, D), lambda i: (i, 0, 0)),
            pl.BlockSpec((H, D), lambda i: (0, 0)),
            pl.BlockSpec((H, D), lambda i: (0, 0)),
        ],
        out_specs=pl.BlockSpec((SROWS // PAGE, H, 1), lambda i: (i, 0, 0)),
        out_shape=jax.ShapeDtypeStruct((P, H, 1), jnp.float32),
        compiler_params=pltpu.CompilerParams(
            dimension_semantics=("arbitrary",)),
        interpret=INTERPRET,
    )(k_cache, knew, q)


def _cut_kernel(est_ref, bias_ref):
    e = est_ref[...]
    i = lax.bitcast_convert_type(e, jnp.int32)
    enc = jnp.where(i >= 0, i, i ^ jnp.int32(0x7FFFFFFF))
    t = jnp.full((1, H), jnp.int32(-2147483648))
    kk = jnp.int32(NSEL)
    cnt0 = jnp.sum((enc >= 0).astype(jnp.int32), axis=0, keepdims=True)
    t = jnp.where(cnt0 >= kk, jnp.zeros_like(t), t)
    for b in range(30, -1, -1):
        cand = t + jnp.int32(1 << b)
        cnt = jnp.sum((enc >= cand).astype(jnp.int32), axis=0, keepdims=True)
        t = jnp.where(cnt >= kk, cand, t)
    gt = enc > t
    cnt_gt = jnp.sum(gt.astype(jnp.int32), axis=0, keepdims=True)
    need = (kk - cnt_gt).astype(jnp.float32)
    eq = enc == t
    eqf = eq.astype(jnp.float32)
    r = lax.broadcasted_iota(jnp.int32, (P, P), 0)
    c = lax.broadcasted_iota(jnp.int32, (P, P), 1)
    lt = (c <= r).astype(jnp.float32)
    prefix = lax.dot_general(lt, eqf, (((1,), (0,)), ((), ())),
                             preferred_element_type=jnp.float32)
    take = jnp.logical_or(gt, jnp.logical_and(eq, prefix <= need))
    bias_ref[...] = jnp.where(take, 0.0, NEG)


def _page_cut(est):
    return pl.pallas_call(
        _cut_kernel,
        in_specs=[pl.BlockSpec((P, H), lambda: (0, 0))],
        out_specs=pl.BlockSpec((P, H), lambda: (0, 0)),
        out_shape=jax.ShapeDtypeStruct((P, H), jnp.float32),
        interpret=INTERPRET,
    )(est)


_SC_L = 16


def _sc_cut_body(bits_hbm, bias_hbm, bits_v, enc_v, bias_v):
    wid = lax.axis_index("s") * 2 + lax.axis_index("c")
    pltpu.sync_copy(bits_hbm.at[wid], bits_v)
    nchunk = P // _SC_L
    one = jnp.ones((_SC_L,), jnp.int32)
    zer = jnp.zeros((_SC_L,), jnp.int32)

    def enc_chunk(j, _):
        i = bits_v[pl.ds(j * _SC_L, _SC_L)]
        enc_v[pl.ds(j * _SC_L, _SC_L)] = jnp.where(
            i >= 0, i, i ^ jnp.int32(0x7FFFFFFF))
        return 0

    lax.fori_loop(0, nchunk, enc_chunk, 0)

    def count_ge(cand_s):
        cand = jnp.full((_SC_L,), cand_s, jnp.int32)

        def body(j, acc):
            e = enc_v[pl.ds(j * _SC_L, _SC_L)]
            return acc + jnp.where(e >= cand, one, zer)

        acc = lax.fori_loop(0, nchunk, body, jnp.zeros((_SC_L,), jnp.int32))
        tot = jnp.int32(0)
        for l in range(_SC_L):
            tot = tot + acc[l]
        return tot

    kk = jnp.int32(NSEL)
    t = jnp.where(count_ge(jnp.int32(0)) >= kk,
                  jnp.int32(0), jnp.int32(-2147483648))
    for b in range(30, -1, -1):
        cand = t + jnp.int32(1 << b)
        t = jnp.where(count_ge(cand) >= kk, cand, t)
    need = kk - count_ge(t + jnp.int32(1))

    tvec = jnp.full((_SC_L,), t, jnp.int32)
    nvec = jnp.full((_SC_L,), need, jnp.int32)
    ramp = lax.iota(jnp.int32, _SC_L)
    ties = jnp.int32(0)
    for j in range(nchunk):
        e = enc_v[pl.ds(j * _SC_L, _SC_L)]
        eq = e == tvec
        eqi = jnp.where(eq, one, zer)
        pre = jnp.full((_SC_L,), ties, jnp.int32)
        for l in range(_SC_L):
            pre = pre + jnp.where(ramp >= l, eqi[l], jnp.int32(0))
        ties = pre[_SC_L - 1]
        take = jnp.logical_or(e > tvec, jnp.logical_and(eq, pre <= nvec))
        bias_v[pl.ds(j * _SC_L, _SC_L)] = jnp.where(take, 0.0, NEG)
    pltpu.sync_copy(bias_v, bias_hbm.at[wid])


def _page_cut_sc(est_hp):
    bits = lax.bitcast_convert_type(est_hp, jnp.int32)
    fn = functools.partial(
        pl.kernel,
        mesh=plsc.VectorSubcoreMesh(core_axis_name="c", subcore_axis_name="s"),
        out_type=jax.ShapeDtypeStruct((H, P), jnp.float32),
        scratch_types=[
            pltpu.VMEM((P,), jnp.int32),
            pltpu.VMEM((P,), jnp.int32),
            pltpu.VMEM((P,), jnp.float32),
        ],
    )(_sc_cut_body)
    return fn(bits)


def _attn_kernel(k_ref, v_ref, q_ref, knew_ref, vnew_ref, bias_ref, o_ref,
                 acc_ref, m_ref, l_ref):
    i = pl.program_id(0)

    @pl.when(i == 0)
    def _init():
        m_ref[...] = jnp.full((H, 1), NEG)
        l_ref[...] = jnp.zeros((H, 1), jnp.float32)
        acc_ref[...] = jnp.zeros((H, D), jnp.float32)

    kblk = k_ref[...]
    vblk = v_ref[...]
    row = i * SROWS + lax.broadcasted_iota(jnp.int32, (SROWS, 1, 1), 0)
    isnew = row == SEQ_PREV
    kblk = jnp.where(isnew, knew_ref[...][None], kblk)
    vblk = jnp.where(isnew, vnew_ref[...][None], vblk)

    q = q_ref[...]
    s = jnp.sum(kblk * q, axis=-1)
    s = s * (1.0 / (D ** 0.5)) + bias_ref[...]
    m_prev = m_ref[...]
    m_new = jnp.maximum(m_prev, jnp.max(s, axis=0)[:, None])
    corr = jnp.exp(m_prev - m_new)
    p = jnp.exp(s - m_new[:, 0])
    l_ref[...] = l_ref[...] * corr + jnp.sum(p, axis=0)[:, None]
    acc_ref[...] = acc_ref[...] * corr + jnp.sum(p[:, :, None] * vblk, axis=0)
    m_ref[...] = m_new

    @pl.when(i == NSLAB - 1)
    def _fin():
        o_ref[...] = acc_ref[...] / l_ref[...]


def _masked_attn(k_cache, v_cache, q, knew, vnew, bias_rows):
    return pl.pallas_call(
        _attn_kernel,
        grid=(NSLAB,),
        in_specs=[
            pl.BlockSpec((SROWS, H, D), lambda i: (i, 0, 0)),
            pl.BlockSpec((SROWS, H, D), lambda i: (i, 0, 0)),
            pl.BlockSpec((H, D), lambda i: (0, 0)),
            pl.BlockSpec((H, D), lambda i: (0, 0)),
            pl.BlockSpec((H, D), lambda i: (0, 0)),
            pl.BlockSpec((SROWS, H), lambda i: (i, 0)),
        ],
        out_specs=pl.BlockSpec((H, D), lambda i: (0, 0)),
        out_shape=jax.ShapeDtypeStruct((H, D), jnp.float32),
        scratch_shapes=[
            pltpu.VMEM((H, D), jnp.float32),
            pltpu.VMEM((H, 1), jnp.float32),
            pltpu.VMEM((H, 1), jnp.float32),
        ],
        compiler_params=pltpu.CompilerParams(
            dimension_semantics=("arbitrary",)),
        interpret=INTERPRET,
    )(k_cache, v_cache, q, knew, vnew, bias_rows)


def kernel(hidden_states, k_cache, v_cache, Wq, Wk, Wv, Wo):
    x = hidden_states.reshape(1, HID)
    d2 = D // 2
    inv_freq = 1.0 / (ROPE_THETA ** (jnp.arange(0, d2, dtype=jnp.float32) / d2))
    ang = jnp.float32(SEQ_PREV) * inv_freq
    cos = jnp.cos(ang).reshape(1, d2)
    sin = jnp.sin(ang).reshape(1, d2)

    q = _proj(x, Wq, cos, sin).reshape(H, D)
    knew = _proj(x, Wk, cos, sin).reshape(H, D)
    vnew = _proj(x, Wv).reshape(H, D)

    est = _estimate(k_cache, knew, q)
    bias = _page_cut_sc(est.reshape(P, H).T).T
    bias_rows = jnp.repeat(bias, PAGE, axis=0)

    att = _masked_attn(k_cache, v_cache, q, knew, vnew, bias_rows)

    out = _proj(att.reshape(1, HID), Wo)
    return out.reshape(1, 1, HID)


# speedup vs baseline: 1.6651x; 1.0331x over previous
import functools

import jax
import jax.numpy as jnp
from jax import lax
from jax.experimental import pallas as pl
from jax.experimental.pallas import tpu as pltpu
from jax.experimental.pallas import tpu_sc as plsc

H = 32
D = 128
HID = 4096
SEQ_PREV = 8191
PAGE = 16
BUDGET = 2048
ROPE_THETA = 10000.0
P = (SEQ_PREV + 1) // PAGE
NSEL = BUDGET // PAGE
SROWS = 512
NSLAB = (SEQ_PREV + 1) // SROWS
NEG = -1e30

INTERPRET = False


def _proj_kernel_rope(x_ref, w_ref, cos_ref, sin_ref, o_ref):
    t = lax.dot_general(x_ref[...], w_ref[...], (((1,), (1,)), ((), ())),
                        preferred_element_type=jnp.float32)
    c = cos_ref[...]
    s = sin_ref[...]
    x1 = t[:, : D // 2]
    x2 = t[:, D // 2:]
    o_ref[0] = jnp.concatenate([x1 * c - x2 * s, x2 * c + x1 * s], axis=1)


def _proj_kernel_plain(x_ref, w_ref, o_ref):
    o_ref[0] = lax.dot_general(x_ref[...], w_ref[...], (((1,), (1,)), ((), ())),
                               preferred_element_type=jnp.float32)


def _proj(x, w, cos=None, sin=None):
    rope = cos is not None
    in_specs = [
        pl.BlockSpec((1, HID), lambda i: (0, 0)),
        pl.BlockSpec((D, HID), lambda i: (i, 0)),
    ]
    args = [x, w]
    if rope:
        in_specs += [pl.BlockSpec((1, D // 2), lambda i: (0, 0)),
                     pl.BlockSpec((1, D // 2), lambda i: (0, 0))]
        args += [cos, sin]
    return pl.pallas_call(
        _proj_kernel_rope if rope else _proj_kernel_plain,
        grid=(HID // D,),
        in_specs=in_specs,
        out_specs=pl.BlockSpec((1, 1, D), lambda i: (i, 0, 0)),
        out_shape=jax.ShapeDtypeStruct((HID // D, 1, D), jnp.float32),
        interpret=INTERPRET,
    )(*args)


def _est_kernel(k_ref, knew_ref, q_ref, est_ref):
    sb = pl.program_id(0)
    blk = k_ref[...]
    row = sb * SROWS + lax.broadcasted_iota(jnp.int32, (SROWS, 1, 1), 0)
    blk = jnp.where(row == SEQ_PREV, knew_ref[...][None], blk)
    pages = blk.reshape(SROWS // PAGE, PAGE, H, D)
    kmin = pages.min(axis=1)
    kmax = pages.max(axis=1)
    q = q_ref[...]
    m = jnp.maximum(q * kmin, q * kmax)
    est_ref[...] = jnp.sum(m, axis=-1, keepdims=True)


def _estimate(k_cache, knew, q):
    return pl.pallas_call(
        _est_kernel,
        grid=(NSLAB,),
        in_specs=[
            pl.BlockSpec((SROWS, H, D), lambda i: (i, 0, 0)),
            pl.BlockSpec((H, D), lambda i: (0, 0)),
            pl.BlockSpec((H, D), lambda i: (0, 0)),
        ],
        out_specs=pl.BlockSpec((SROWS // PAGE, H, 1), lambda i: (i, 0, 0)),
        out_shape=jax.ShapeDtypeStruct((P, H, 1), jnp.float32),
        compiler_params=pltpu.CompilerParams(
            dimension_semantics=("arbitrary",)),
        interpret=INTERPRET,
    )(k_cache, knew, q)


def _cut_kernel(est_ref, bias_ref):
    e = est_ref[...]
    i = lax.bitcast_convert_type(e, jnp.int32)
    enc = jnp.where(i >= 0, i, i ^ jnp.int32(0x7FFFFFFF))
    t = jnp.full((1, H), jnp.int32(-2147483648))
    kk = jnp.int32(NSEL)
    cnt0 = jnp.sum((enc >= 0).astype(jnp.int32), axis=0, keepdims=True)
    t = jnp.where(cnt0 >= kk, jnp.zeros_like(t), t)
    for b in range(30, -1, -1):
        cand = t + jnp.int32(1 << b)
        cnt = jnp.sum((enc >= cand).astype(jnp.int32), axis=0, keepdims=True)
        t = jnp.where(cnt >= kk, cand, t)
    gt = enc > t
    cnt_gt = jnp.sum(gt.astype(jnp.int32), axis=0, keepdims=True)
    need = (kk - cnt_gt).astype(jnp.float32)
    eq = enc == t
    eqf = eq.astype(jnp.float32)
    r = lax.broadcasted_iota(jnp.int32, (P, P), 0)
    c = lax.broadcasted_iota(jnp.int32, (P, P), 1)
    lt = (c <= r).astype(jnp.float32)
    prefix = lax.dot_general(lt, eqf, (((1,), (0,)), ((), ())),
                             preferred_element_type=jnp.float32)
    take = jnp.logical_or(gt, jnp.logical_and(eq, prefix <= need))
    bias_ref[...] = jnp.where(take, 0.0, NEG)


def _page_cut(est):
    return pl.pallas_call(
        _cut_kernel,
        in_specs=[pl.BlockSpec((P, H), lambda: (0, 0))],
        out_specs=pl.BlockSpec((P, H), lambda: (0, 0)),
        out_shape=jax.ShapeDtypeStruct((P, H), jnp.float32),
        interpret=INTERPRET,
    )(est)


_SC_L = 16


def _sc_cut_body(bits_hbm, bias_hbm, bits_v, enc_v, bias_v):
    wid = lax.axis_index("s") * 2 + lax.axis_index("c")
    pltpu.sync_copy(bits_hbm.at[wid], bits_v)
    nchunk = P // _SC_L
    one = jnp.ones((_SC_L,), jnp.int32)
    zer = jnp.zeros((_SC_L,), jnp.int32)

    def enc_chunk(j, _):
        i = bits_v[pl.ds(j * _SC_L, _SC_L)]
        enc_v[pl.ds(j * _SC_L, _SC_L)] = jnp.where(
            i >= 0, i, i ^ jnp.int32(0x7FFFFFFF))
        return 0

    lax.fori_loop(0, nchunk, enc_chunk, 0)

    def count_ge(cand_s):
        cand = jnp.full((_SC_L,), cand_s, jnp.int32)

        def body(j, acc):
            e = enc_v[pl.ds(j * _SC_L, _SC_L)]
            return acc + jnp.where(e >= cand, one, zer)

        acc = lax.fori_loop(0, nchunk, body, jnp.zeros((_SC_L,), jnp.int32))
        tot = jnp.int32(0)
        for l in range(_SC_L):
            tot = tot + acc[l]
        return tot

    kk = jnp.int32(NSEL)
    t = jnp.where(count_ge(jnp.int32(0)) >= kk,
                  jnp.int32(0), jnp.int32(-2147483648))
    for b in range(30, -1, -1):
        cand = t + jnp.int32(1 << b)
        t = jnp.where(count_ge(cand) >= kk, cand, t)
    need = kk - count_ge(t + jnp.int32(1))

    tvec = jnp.full((_SC_L,), t, jnp.int32)
    nvec = jnp.full((_SC_L,), need, jnp.int32)
    ramp = lax.iota(jnp.int32, _SC_L)
    ties = jnp.int32(0)
    for j in range(nchunk):
        e = enc_v[pl.ds(j * _SC_L, _SC_L)]
        eq = e == tvec
        eqi = jnp.where(eq, one, zer)
        pre = jnp.full((_SC_L,), ties, jnp.int32)
        for l in range(_SC_L):
            pre = pre + jnp.where(ramp >= l, eqi[l], jnp.int32(0))
        ties = pre[_SC_L - 1]
        take = jnp.logical_or(e > tvec, jnp.logical_and(eq, pre <= nvec))
        bias_v[pl.ds(j * _SC_L, _SC_L)] = jnp.where(take, 0.0, NEG)
    pltpu.sync_copy(bias_v, bias_hbm.at[wid])


def _page_cut_sc(est_hp):
    bits = lax.bitcast_convert_type(est_hp, jnp.int32)
    fn = functools.partial(
        pl.kernel,
        mesh=plsc.VectorSubcoreMesh(core_axis_name="c", subcore_axis_name="s"),
        out_type=jax.ShapeDtypeStruct((H, P), jnp.float32),
        scratch_types=[
            pltpu.VMEM((P,), jnp.int32),
            pltpu.VMEM((P,), jnp.int32),
            pltpu.VMEM((P,), jnp.float32),
        ],
    )(_sc_cut_body)
    return fn(bits)


def _attn_kernel(k_ref, v_ref, q_ref, knew_ref, vnew_ref, bias_ref, o_ref,
                 acc_ref, m_ref, l_ref):
    i = pl.program_id(0)

    @pl.when(i == 0)
    def _init():
        m_ref[...] = jnp.full((H, 1), NEG)
        l_ref[...] = jnp.zeros((H, 1), jnp.float32)
        acc_ref[...] = jnp.zeros((H, D), jnp.float32)

    kblk = k_ref[...]
    vblk = v_ref[...]
    row = i * SROWS + lax.broadcasted_iota(jnp.int32, (SROWS, 1, 1), 0)
    isnew = row == SEQ_PREV
    kblk = jnp.where(isnew, knew_ref[...][None], kblk)
    vblk = jnp.where(isnew, vnew_ref[...][None], vblk)

    q = q_ref[...]
    s = jnp.sum(kblk * q, axis=-1)
    s = s * (1.0 / (D ** 0.5)) + bias_ref[...]
    m_prev = m_ref[...]
    m_new = jnp.maximum(m_prev, jnp.max(s, axis=0)[:, None])
    corr = jnp.exp(m_prev - m_new)
    p = jnp.exp(s - m_new[:, 0])
    l_ref[...] = l_ref[...] * corr + jnp.sum(p, axis=0)[:, None]
    acc_ref[...] = acc_ref[...] * corr + jnp.sum(p[:, :, None] * vblk, axis=0)
    m_ref[...] = m_new

    @pl.when(i == NSLAB - 1)
    def _fin():
        o_ref[...] = acc_ref[...] / l_ref[...]


def _masked_attn(k_cache, v_cache, q, knew, vnew, bias_rows):
    return pl.pallas_call(
        _attn_kernel,
        grid=(NSLAB,),
        in_specs=[
            pl.BlockSpec((SROWS, H, D), lambda i: (i, 0, 0)),
            pl.BlockSpec((SROWS, H, D), lambda i: (i, 0, 0)),
            pl.BlockSpec((H, D), lambda i: (0, 0)),
            pl.BlockSpec((H, D), lambda i: (0, 0)),
            pl.BlockSpec((H, D), lambda i: (0, 0)),
            pl.BlockSpec((SROWS, H), lambda i: (i, 0)),
        ],
        out_specs=pl.BlockSpec((H, D), lambda i: (0, 0)),
        out_shape=jax.ShapeDtypeStruct((H, D), jnp.float32),
        scratch_shapes=[
            pltpu.VMEM((H, D), jnp.float32),
            pltpu.VMEM((H, 1), jnp.float32),
            pltpu.VMEM((H, 1), jnp.float32),
        ],
        compiler_params=pltpu.CompilerParams(
            dimension_semantics=("arbitrary",)),
        interpret=INTERPRET,
    )(k_cache, v_cache, q, knew, vnew, bias_rows)


def kernel(hidden_states, k_cache, v_cache, Wq, Wk, Wv, Wo):
    x = hidden_states.reshape(1, HID)
    d2 = D // 2
    inv_freq = 1.0 / (ROPE_THETA ** (jnp.arange(0, d2, dtype=jnp.float32) / d2))
    ang = jnp.float32(SEQ_PREV) * inv_freq
    cos = jnp.cos(ang).reshape(1, d2)
    sin = jnp.sin(ang).reshape(1, d2)

    q = _proj(x, Wq, cos, sin).reshape(H, D)
    knew = _proj(x, Wk, cos, sin).reshape(H, D)
    vnew = _proj(x, Wv).reshape(H, D)

    est = _estimate(k_cache, knew, q)
    bias = _page_cut_sc(est.reshape(P, H).T).T
    bias_rows = jnp.repeat(bias, PAGE, axis=0)

    att = _masked_attn(k_cache, v_cache, q, knew, vnew, bias_rows)

    out = _proj(att.reshape(1, HID), Wo)
    return out.reshape(1, 1, HID)


# speedup vs baseline: 1.7682x; 1.0619x over previous
import functools

import jax
import jax.numpy as jnp
from jax import lax
from jax.experimental import pallas as pl
from jax.experimental.pallas import tpu as pltpu
from jax.experimental.pallas import tpu_sc as plsc

H = 32
D = 128
HID = 4096
SEQ_PREV = 8191
PAGE = 16
BUDGET = 2048
ROPE_THETA = 10000.0
P = (SEQ_PREV + 1) // PAGE
NSEL = BUDGET // PAGE
SROWS = 512
NSLAB = (SEQ_PREV + 1) // SROWS
NEG = -1e30

INTERPRET = False


def _proj_kernel_rope(x_ref, w_ref, cos_ref, sin_ref, o_ref):
    t = lax.dot_general(x_ref[...], w_ref[...], (((1,), (1,)), ((), ())),
                        preferred_element_type=jnp.float32)
    c = cos_ref[...]
    s = sin_ref[...]
    x1 = t[:, : D // 2]
    x2 = t[:, D // 2:]
    o_ref[0] = jnp.concatenate([x1 * c - x2 * s, x2 * c + x1 * s], axis=1)


def _proj_kernel_plain(x_ref, w_ref, o_ref):
    o_ref[0] = lax.dot_general(x_ref[...], w_ref[...], (((1,), (1,)), ((), ())),
                               preferred_element_type=jnp.float32)


def _proj(x, w, cos=None, sin=None):
    rope = cos is not None
    in_specs = [
        pl.BlockSpec((1, HID), lambda i: (0, 0)),
        pl.BlockSpec((D, HID), lambda i: (i, 0)),
    ]
    args = [x, w]
    if rope:
        in_specs += [pl.BlockSpec((1, D // 2), lambda i: (0, 0)),
                     pl.BlockSpec((1, D // 2), lambda i: (0, 0))]
        args += [cos, sin]
    return pl.pallas_call(
        _proj_kernel_rope if rope else _proj_kernel_plain,
        grid=(HID // D,),
        in_specs=in_specs,
        out_specs=pl.BlockSpec((1, 1, D), lambda i: (i, 0, 0)),
        out_shape=jax.ShapeDtypeStruct((HID // D, 1, D), jnp.float32),
        interpret=INTERPRET,
    )(*args)


EROWS = 1024
NESLAB = (SEQ_PREV + 1) // EROWS


def _est_kernel(k_ref, knew_ref, q_ref, est_ref):
    sb = pl.program_id(0)
    blk = k_ref[...]
    row = sb * EROWS + lax.broadcasted_iota(jnp.int32, (EROWS, 1, 1), 0)
    blk = jnp.where(row == SEQ_PREV, knew_ref[...][None], blk)
    pages = blk.reshape(EROWS // PAGE, PAGE, H, D)
    kmin = pages.min(axis=1)
    kmax = pages.max(axis=1)
    q = q_ref[...]
    m = jnp.maximum(q * kmin, q * kmax)
    est_ref[...] = jnp.sum(m, axis=-1, keepdims=True)


def _estimate(k_cache, knew, q):
    return pl.pallas_call(
        _est_kernel,
        grid=(NESLAB,),
        in_specs=[
            pl.BlockSpec((EROWS, H, D), lambda i: (i, 0, 0)),
            pl.BlockSpec((H, D), lambda i: (0, 0)),
            pl.BlockSpec((H, D), lambda i: (0, 0)),
        ],
        out_specs=pl.BlockSpec((EROWS // PAGE, H, 1), lambda i: (i, 0, 0)),
        out_shape=jax.ShapeDtypeStruct((P, H, 1), jnp.float32),
        compiler_params=pltpu.CompilerParams(
            dimension_semantics=("arbitrary",)),
        interpret=INTERPRET,
    )(k_cache, knew, q)


def _cut_kernel(est_ref, bias_ref):
    e = est_ref[...]
    i = lax.bitcast_convert_type(e, jnp.int32)
    enc = jnp.where(i >= 0, i, i ^ jnp.int32(0x7FFFFFFF))
    t = jnp.full((1, H), jnp.int32(-2147483648))
    kk = jnp.int32(NSEL)
    cnt0 = jnp.sum((enc >= 0).astype(jnp.int32), axis=0, keepdims=True)
    t = jnp.where(cnt0 >= kk, jnp.zeros_like(t), t)
    for b in range(30, -1, -1):
        cand = t + jnp.int32(1 << b)
        cnt = jnp.sum((enc >= cand).astype(jnp.int32), axis=0, keepdims=True)
        t = jnp.where(cnt >= kk, cand, t)
    gt = enc > t
    cnt_gt = jnp.sum(gt.astype(jnp.int32), axis=0, keepdims=True)
    need = (kk - cnt_gt).astype(jnp.float32)
    eq = enc == t
    eqf = eq.astype(jnp.float32)
    r = lax.broadcasted_iota(jnp.int32, (P, P), 0)
    c = lax.broadcasted_iota(jnp.int32, (P, P), 1)
    lt = (c <= r).astype(jnp.float32)
    prefix = lax.dot_general(lt, eqf, (((1,), (0,)), ((), ())),
                             preferred_element_type=jnp.float32)
    take = jnp.logical_or(gt, jnp.logical_and(eq, prefix <= need))
    bias_ref[...] = jnp.where(take, 0.0, NEG)


def _page_cut(est):
    return pl.pallas_call(
        _cut_kernel,
        in_specs=[pl.BlockSpec((P, H), lambda: (0, 0))],
        out_specs=pl.BlockSpec((P, H), lambda: (0, 0)),
        out_shape=jax.ShapeDtypeStruct((P, H), jnp.float32),
        interpret=INTERPRET,
    )(est)


_SC_L = 16


def _sc_cut_body(bits_hbm, bias_hbm, bits_v, enc_v, bias_v):
    wid = lax.axis_index("s") * 2 + lax.axis_index("c")
    pltpu.sync_copy(bits_hbm.at[wid], bits_v)
    nchunk = P // _SC_L
    one = jnp.ones((_SC_L,), jnp.int32)
    zer = jnp.zeros((_SC_L,), jnp.int32)

    def enc_chunk(j, _):
        i = bits_v[pl.ds(j * _SC_L, _SC_L)]
        enc_v[pl.ds(j * _SC_L, _SC_L)] = jnp.where(
            i >= 0, i, i ^ jnp.int32(0x7FFFFFFF))
        return 0

    lax.fori_loop(0, nchunk, enc_chunk, 0)

    def count_ge(cand_s):
        cand = jnp.full((_SC_L,), cand_s, jnp.int32)

        def body(j, acc):
            e = enc_v[pl.ds(j * _SC_L, _SC_L)]
            return acc + jnp.where(e >= cand, one, zer)

        acc = lax.fori_loop(0, nchunk, body, jnp.zeros((_SC_L,), jnp.int32))
        tot = jnp.int32(0)
        for l in range(_SC_L):
            tot = tot + acc[l]
        return tot

    kk = jnp.int32(NSEL)
    t = jnp.where(count_ge(jnp.int32(0)) >= kk,
                  jnp.int32(0), jnp.int32(-2147483648))
    for b in range(30, -1, -1):
        cand = t + jnp.int32(1 << b)
        t = jnp.where(count_ge(cand) >= kk, cand, t)
    need = kk - count_ge(t + jnp.int32(1))

    tvec = jnp.full((_SC_L,), t, jnp.int32)
    nvec = jnp.full((_SC_L,), need, jnp.int32)
    ramp = lax.iota(jnp.int32, _SC_L)
    ties = jnp.int32(0)
    for j in range(nchunk):
        e = enc_v[pl.ds(j * _SC_L, _SC_L)]
        eq = e == tvec
        eqi = jnp.where(eq, one, zer)
        pre = jnp.full((_SC_L,), ties, jnp.int32)
        for l in range(_SC_L):
            pre = pre + jnp.where(ramp >= l, eqi[l], jnp.int32(0))
        ties = pre[_SC_L - 1]
        take = jnp.logical_or(e > tvec, jnp.logical_and(eq, pre <= nvec))
        bias_v[pl.ds(j * _SC_L, _SC_L)] = jnp.where(take, 0.0, NEG)
    pltpu.sync_copy(bias_v, bias_hbm.at[wid])


def _page_cut_sc(est_hp):
    bits = lax.bitcast_convert_type(est_hp, jnp.int32)
    fn = functools.partial(
        pl.kernel,
        mesh=plsc.VectorSubcoreMesh(core_axis_name="c", subcore_axis_name="s"),
        out_type=jax.ShapeDtypeStruct((H, P), jnp.float32),
        scratch_types=[
            pltpu.VMEM((P,), jnp.int32),
            pltpu.VMEM((P,), jnp.int32),
            pltpu.VMEM((P,), jnp.float32),
        ],
    )(_sc_cut_body)
    return fn(bits)


def _attn_kernel(k_ref, v_ref, q_ref, knew_ref, vnew_ref, bias_ref, o_ref,
                 acc_ref, m_ref, l_ref):
    i = pl.program_id(0)

    @pl.when(i == 0)
    def _init():
        m_ref[...] = jnp.full((H, 1), NEG)
        l_ref[...] = jnp.zeros((H, 1), jnp.float32)
        acc_ref[...] = jnp.zeros((H, D), jnp.float32)

    kblk = k_ref[...]
    vblk = v_ref[...]
    row = i * SROWS + lax.broadcasted_iota(jnp.int32, (SROWS, 1, 1), 0)
    isnew = row == SEQ_PREV
    kblk = jnp.where(isnew, knew_ref[...][None], kblk)
    vblk = jnp.where(isnew, vnew_ref[...][None], vblk)

    q = q_ref[...]
    s = jnp.sum(kblk * q, axis=-1)
    pbias = bias_ref[...]
    bias = jnp.broadcast_to(pbias[:, None, :],
                            (SROWS // PAGE, PAGE, H)).reshape(SROWS, H)
    s = s * (1.0 / (D ** 0.5)) + bias
    m_prev = m_ref[...]
    m_new = jnp.maximum(m_prev, jnp.max(s, axis=0)[:, None])
    corr = jnp.exp(m_prev - m_new)
    p = jnp.exp(s - m_new[:, 0])
    l_ref[...] = l_ref[...] * corr + jnp.sum(p, axis=0)[:, None]
    acc_ref[...] = acc_ref[...] * corr + jnp.sum(p[:, :, None] * vblk, axis=0)
    m_ref[...] = m_new

    @pl.when(i == NSLAB - 1)
    def _fin():
        o_ref[...] = acc_ref[...] / l_ref[...]


def _masked_attn(k_cache, v_cache, q, knew, vnew, bias_rows):
    return pl.pallas_call(
        _attn_kernel,
        grid=(NSLAB,),
        in_specs=[
            pl.BlockSpec((SROWS, H, D), lambda i: (i, 0, 0)),
            pl.BlockSpec((SROWS, H, D), lambda i: (i, 0, 0)),
            pl.BlockSpec((H, D), lambda i: (0, 0)),
            pl.BlockSpec((H, D), lambda i: (0, 0)),
            pl.BlockSpec((H, D), lambda i: (0, 0)),
            pl.BlockSpec((SROWS // PAGE, H), lambda i: (i, 0)),
        ],
        out_specs=pl.BlockSpec((H, D), lambda i: (0, 0)),
        out_shape=jax.ShapeDtypeStruct((H, D), jnp.float32),
        scratch_shapes=[
            pltpu.VMEM((H, D), jnp.float32),
            pltpu.VMEM((H, 1), jnp.float32),
            pltpu.VMEM((H, 1), jnp.float32),
        ],
        compiler_params=pltpu.CompilerParams(
            dimension_semantics=("arbitrary",)),
        interpret=INTERPRET,
    )(k_cache, v_cache, q, knew, vnew, bias_rows)


def kernel(hidden_states, k_cache, v_cache, Wq, Wk, Wv, Wo):
    x = hidden_states.reshape(1, HID)
    d2 = D // 2
    inv_freq = 1.0 / (ROPE_THETA ** (jnp.arange(0, d2, dtype=jnp.float32) / d2))
    ang = jnp.float32(SEQ_PREV) * inv_freq
    cos = jnp.cos(ang).reshape(1, d2)
    sin = jnp.sin(ang).reshape(1, d2)

    q = _proj(x, Wq, cos, sin).reshape(H, D)
    knew = _proj(x, Wk, cos, sin).reshape(H, D)
    vnew = _proj(x, Wv).reshape(H, D)

    est = _estimate(k_cache, knew, q)
    bias = _page_cut_sc(est.reshape(P, H).T).T

    att = _masked_attn(k_cache, v_cache, q, knew, vnew, bias)

    out = _proj(att.reshape(1, HID), Wo)
    return out.reshape(1, 1, HID)
